# baseline (speedup 1.0000x reference)
import functools

import jax
import jax.numpy as jnp
from jax import lax
from jax.experimental import pallas as pl
from jax.experimental.pallas import tpu as pltpu
from jax.experimental.pallas import tpu_sc as plsc

VOCAB = 1000000
EMBED_DIM = 32
SEQ_LEN = 200
BATCH = 4096

_INFO = plsc.get_sparse_core_info()
_NC = _INFO.num_cores
_NS = _INFO.num_subcores
_NW = _NC * _NS
_BPW = BATCH // _NW
_L = 16
_NBUF = 8
_NGRP = SEQ_LEN // _NBUF


def _embed_bag_body(tok_hbm, table_hbm, out_hbm, idx_all, rows_v, acc_v,
                    *sems):
    wid = lax.axis_index("s") * _NC + lax.axis_index("c")

    pltpu.sync_copy(tok_hbm.at[wid], idx_all)

    zeros = jnp.zeros((_L,), jnp.float32)

    def zbody(b, _):
        acc_v[b, pl.ds(0, _L)] = zeros
        acc_v[b, pl.ds(_L, _L)] = zeros
        return 0
    lax.fori_loop(0, _BPW, zbody, 0, unroll=8)

    def fire(s, b):
        pltpu.async_copy(table_hbm.at[idx_all.at[s]], rows_v.at[b], sems[b])

    for b in range(_NBUF):
        fire(b, b)

    def grp(g, _):
        for b in range(_NBUF):
            s = g * _NBUF + b
            pltpu.make_async_copy(
                table_hbm.at[idx_all.at[s]], rows_v.at[b], sems[b]).wait()

            def abody(r, _, b=b):
                r0 = rows_v[b, r, pl.ds(0, _L)]
                r1 = rows_v[b, r, pl.ds(_L, _L)]
                plsc.addupdate(acc_v.at[r, pl.ds(0, _L)], r0)
                plsc.addupdate(acc_v.at[r, pl.ds(_L, _L)], r1)
                return 0
            lax.fori_loop(0, _BPW, abody, 0, unroll=8)

            nxt = s + _NBUF

            @pl.when(nxt < SEQ_LEN)
            def _(nxt=nxt, b=b):
                fire(nxt, b)
        return 0
    lax.fori_loop(0, _NGRP, grp, 0)

    pltpu.sync_copy(acc_v, out_hbm.at[pl.ds(wid * _BPW, _BPW)])


def _detile_body(tok_ref, out_ref):
    out_ref[0] = tok_ref[...]


@jax.jit
def kernel(sentence_tokens, embedding_table):
    tok = sentence_tokens.astype(jnp.int32)
    tok = pl.pallas_call(
        _detile_body,
        grid=(_NW,),
        in_specs=[pl.BlockSpec((SEQ_LEN, _BPW), lambda j: (0, j))],
        out_specs=pl.BlockSpec((1, SEQ_LEN, _BPW), lambda j: (j, 0, 0)),
        out_shape=jax.ShapeDtypeStruct((_NW, SEQ_LEN, _BPW), jnp.int32),
    )(tok)
    mesh = plsc.VectorSubcoreMesh(core_axis_name="c", subcore_axis_name="s")
    run = functools.partial(
        pl.kernel,
        out_type=jax.ShapeDtypeStruct((BATCH, EMBED_DIM), jnp.float32),
        mesh=mesh,
        scratch_types=[
            pltpu.VMEM((SEQ_LEN, _BPW), jnp.int32),
            pltpu.VMEM((_NBUF, _BPW, EMBED_DIM), jnp.float32),
            pltpu.VMEM((_BPW, EMBED_DIM), jnp.float32),
        ] + [pltpu.SemaphoreType.DMA] * _NBUF,
        compiler_params=pltpu.CompilerParams(use_tc_tiling_on_sc=False),
    )(_embed_bag_body)
    return run(tok, embedding_table)


# speedup vs baseline: 1.6481x; 1.6481x over previous
import functools

import jax
import jax.numpy as jnp
from jax import lax
from jax.experimental import pallas as pl
from jax.experimental.pallas import tpu as pltpu
from jax.experimental.pallas import tpu_sc as plsc

VOCAB = 1000000
EMBED_DIM = 32
SEQ_LEN = 200
BATCH = 4096

_INFO = plsc.get_sparse_core_info()
_NC = _INFO.num_cores
_NS = _INFO.num_subcores
_NW = _NC * _NS
_BPW = BATCH // _NW
_L = 16
_NBUF = 8
_NGRP = SEQ_LEN // _NBUF


def _embed_bag_body(tok_hbm, table_hbm, out_hbm, idx_all, rows_v, acc_v,
                    *sems):
    wid = lax.axis_index("s") * _NC + lax.axis_index("c")

    pltpu.sync_copy(tok_hbm.at[wid], idx_all)

    zeros = jnp.zeros((_L,), jnp.float32)

    def zbody(b, _):
        acc_v[b, pl.ds(0, _L)] = zeros
        acc_v[b, pl.ds(_L, _L)] = zeros
        return 0
    lax.fori_loop(0, _BPW, zbody, 0, unroll=8)

    def fire(s, b):
        pltpu.async_copy(table_hbm.at[idx_all.at[s]], rows_v.at[b], sems[b])

    for b in range(_NBUF):
        fire(b, b)

    def grp(g, _):
        for b in range(_NBUF):
            s = g * _NBUF + b
            pltpu.make_async_copy(
                table_hbm.at[idx_all.at[s]], rows_v.at[b], sems[b]).wait()

            def abody(r, _, b=b):
                r0 = rows_v[b, r, pl.ds(0, _L)]
                r1 = rows_v[b, r, pl.ds(_L, _L)]
                plsc.addupdate(acc_v.at[r, pl.ds(0, _L)], r0)
                plsc.addupdate(acc_v.at[r, pl.ds(_L, _L)], r1)
                return 0
            lax.fori_loop(0, _BPW, abody, 0, unroll=8)

            nxt = s + _NBUF

            @pl.when(nxt < SEQ_LEN)
            def _(nxt=nxt, b=b):
                fire(nxt, b)
        return 0
    lax.fori_loop(0, _NGRP, grp, 0)

    pltpu.sync_copy(acc_v, out_hbm.at[pl.ds(wid * _BPW, _BPW)])


def _detile_body(tok_ref, out_ref):
    t = tok_ref[...]
    out_ref[0] = ((t >> 13) * 8192) + ((t & 2047) * 4) + ((t >> 11) & 3)


_TBLK = 8192


_TGRID = pl.cdiv(VOCAB, _TBLK)
_VPAD = _TGRID * _TBLK


def _transpose_body(tab_ref, out_ref):
    y = tab_ref[...].T
    parts = [y[c * (_TBLK // 4):(c + 1) * (_TBLK // 4)] for c in range(4)]
    out_ref[...] = jnp.concatenate(parts, axis=1)


@jax.jit
def kernel(sentence_tokens, embedding_table):
    tok = sentence_tokens.astype(jnp.int32)
    tok = pl.pallas_call(
        _detile_body,
        grid=(_NW,),
        in_specs=[pl.BlockSpec((SEQ_LEN, _BPW), lambda j: (0, j))],
        out_specs=pl.BlockSpec((1, SEQ_LEN, _BPW), lambda j: (j, 0, 0)),
        out_shape=jax.ShapeDtypeStruct((_NW, SEQ_LEN, _BPW), jnp.int32),
    )(tok)
    tab_flat = pl.pallas_call(
        _transpose_body,
        grid=(_TGRID,),
        in_specs=[pl.BlockSpec((EMBED_DIM, _TBLK), lambda j: (0, j))],
        out_specs=pl.BlockSpec((_TBLK // 4, 128), lambda j: (j, 0)),
        out_shape=jax.ShapeDtypeStruct((_VPAD * EMBED_DIM // 128, 128),
                                       jnp.float32),
    )(embedding_table.T)
    table_lin = tab_flat.reshape(_VPAD, EMBED_DIM)
    mesh = plsc.VectorSubcoreMesh(core_axis_name="c", subcore_axis_name="s")
    run = functools.partial(
        pl.kernel,
        out_type=jax.ShapeDtypeStruct((BATCH, EMBED_DIM), jnp.float32),
        mesh=mesh,
        scratch_types=[
            pltpu.VMEM((SEQ_LEN, _BPW), jnp.int32),
            pltpu.VMEM((_NBUF, _BPW, EMBED_DIM), jnp.float32),
            pltpu.VMEM((_BPW, EMBED_DIM), jnp.float32),
        ] + [pltpu.SemaphoreType.DMA] * _NBUF,
        compiler_params=pltpu.CompilerParams(use_tc_tiling_on_sc=False),
    )(_embed_bag_body)
    return run(tok, table_lin)


# speedup vs baseline: 1.9831x; 1.2033x over previous
import functools

import jax
import jax.numpy as jnp
from jax import lax
from jax.experimental import pallas as pl
from jax.experimental.pallas import tpu as pltpu
from jax.experimental.pallas import tpu_sc as plsc

VOCAB = 1000000
EMBED_DIM = 32
SEQ_LEN = 200
BATCH = 4096

_INFO = plsc.get_sparse_core_info()
_NC = _INFO.num_cores
_NS = _INFO.num_subcores
_NW = _NC * _NS
_BPW = BATCH // _NW
_L = 16
_NBUF = 8
_NGRP = SEQ_LEN // _NBUF


def _embed_bag_body(tok_hbm, table_hbm, out_hbm, idx_all, rows_v, acc_v,
                    *sems):
    wid = lax.axis_index("s") * _NC + lax.axis_index("c")

    pltpu.sync_copy(tok_hbm.at[wid], idx_all)

    zeros = jnp.zeros((_L,), jnp.float32)

    def zbody(b, _):
        acc_v[b, pl.ds(0, _L)] = zeros
        acc_v[b, pl.ds(_L, _L)] = zeros
        return 0
    lax.fori_loop(0, _BPW, zbody, 0, unroll=8)

    def fire(s, b):
        pltpu.async_copy(table_hbm.at[idx_all.at[s]], rows_v.at[b], sems[b])

    for b in range(_NBUF):
        fire(b, b)

    def grp(g, _):
        for b in range(_NBUF):
            s = g * _NBUF + b
            pltpu.make_async_copy(
                table_hbm.at[idx_all.at[s]], rows_v.at[b], sems[b]).wait()

            def abody(r, _, b=b):
                r0 = rows_v[b, r, pl.ds(0, _L)]
                r1 = rows_v[b, r, pl.ds(_L, _L)]
                plsc.addupdate(acc_v.at[r, pl.ds(0, _L)], r0)
                plsc.addupdate(acc_v.at[r, pl.ds(_L, _L)], r1)
                return 0
            lax.fori_loop(0, _BPW, abody, 0, unroll=8)

            nxt = s + _NBUF

            @pl.when(nxt < SEQ_LEN)
            def _(nxt=nxt, b=b):
                fire(nxt, b)
        return 0
    lax.fori_loop(0, _NGRP, grp, 0)

    pltpu.sync_copy(acc_v, out_hbm.at[pl.ds(wid * _BPW, _BPW)])


def _detile_body(tok_ref, out_ref):
    t = tok_ref[...]
    out_ref[0] = ((t >> 13) * 8192) + ((t & 2047) * 4) + ((t >> 11) & 3)


_TBLK = 8192
_TCH = 128


_TGRID = pl.cdiv(VOCAB, _TBLK)
_VPAD = _TGRID * _TBLK


def _transpose_body(tab_ref, out_ref):
    q = _TBLK // 4
    x = tab_ref[...]
    eye = jnp.eye(_TCH, dtype=jnp.float32)
    parts = []
    for c in range(4):
        chunks = []
        for k in range(q // _TCH):
            xc = x[:, c * q + k * _TCH:c * q + (k + 1) * _TCH]
            yc = lax.dot_general(eye, xc, (((1,), (1,)), ((), ())))
            chunks.append(yc)
        parts.append(jnp.concatenate(chunks, axis=0))
    out_ref[...] = jnp.concatenate(parts, axis=1)


@jax.jit
def kernel(sentence_tokens, embedding_table):
    tok = sentence_tokens.astype(jnp.int32)
    tok = pl.pallas_call(
        _detile_body,
        grid=(_NW,),
        in_specs=[pl.BlockSpec((SEQ_LEN, _BPW), lambda j: (0, j))],
        out_specs=pl.BlockSpec((1, SEQ_LEN, _BPW), lambda j: (j, 0, 0)),
        out_shape=jax.ShapeDtypeStruct((_NW, SEQ_LEN, _BPW), jnp.int32),
    )(tok)
    tab_flat = pl.pallas_call(
        _transpose_body,
        grid=(_TGRID,),
        in_specs=[pl.BlockSpec((EMBED_DIM, _TBLK), lambda j: (0, j))],
        out_specs=pl.BlockSpec((_TBLK // 4, 128), lambda j: (j, 0)),
        out_shape=jax.ShapeDtypeStruct((_VPAD * EMBED_DIM // 128, 128),
                                       jnp.float32),
    )(embedding_table.T)
    table_lin = tab_flat.reshape(_VPAD, EMBED_DIM)
    mesh = plsc.VectorSubcoreMesh(core_axis_name="c", subcore_axis_name="s")
    run = functools.partial(
        pl.kernel,
        out_type=jax.ShapeDtypeStruct((BATCH, EMBED_DIM), jnp.float32),
        mesh=mesh,
        scratch_types=[
            pltpu.VMEM((SEQ_LEN, _BPW), jnp.int32),
            pltpu.VMEM((_NBUF, _BPW, EMBED_DIM), jnp.float32),
            pltpu.VMEM((_BPW, EMBED_DIM), jnp.float32),
        ] + [pltpu.SemaphoreType.DMA] * _NBUF,
        compiler_params=pltpu.CompilerParams(use_tc_tiling_on_sc=False),
    )(_embed_bag_body)
    return run(tok, table_lin)


# speedup vs baseline: 2.3169x; 1.1683x over previous
import functools

import jax
import jax.numpy as jnp
from jax import lax
from jax.experimental import pallas as pl
from jax.experimental.pallas import tpu as pltpu
from jax.experimental.pallas import tpu_sc as plsc

VOCAB = 1000000
EMBED_DIM = 32
SEQ_LEN = 200
BATCH = 4096

_INFO = plsc.get_sparse_core_info()
_NC = _INFO.num_cores
_NS = _INFO.num_subcores
_NW = _NC * _NS
_BPW = BATCH // _NW
_L = 16
_NBUF = 8
_NGRP = SEQ_LEN // _NBUF


def _embed_bag_body(tok_hbm, table_hbm, out_hbm, idx_all, rows_v, acc_v,
                    *sems):
    wid = lax.axis_index("s") * _NC + lax.axis_index("c")

    pltpu.sync_copy(tok_hbm.at[wid], idx_all)

    zeros = jnp.zeros((_L,), jnp.float32)

    def zbody(b, _):
        acc_v[b, pl.ds(0, _L)] = zeros
        acc_v[b, pl.ds(_L, _L)] = zeros
        return 0
    lax.fori_loop(0, _BPW, zbody, 0, unroll=8)

    def fire(s, b):
        pltpu.async_copy(table_hbm.at[idx_all.at[s]], rows_v.at[b], sems[b])

    for b in range(_NBUF):
        fire(b, b)

    def grp(g, _):
        for b in range(_NBUF):
            s = g * _NBUF + b
            pltpu.make_async_copy(
                table_hbm.at[idx_all.at[s]], rows_v.at[b], sems[b]).wait()

            def abody(r, _, b=b):
                r0 = rows_v[b, r, pl.ds(0, _L)]
                r1 = rows_v[b, r, pl.ds(_L, _L)]
                plsc.addupdate(acc_v.at[r, pl.ds(0, _L)], r0)
                plsc.addupdate(acc_v.at[r, pl.ds(_L, _L)], r1)
                return 0
            lax.fori_loop(0, _BPW, abody, 0, unroll=8)

            nxt = s + _NBUF

            @pl.when(nxt < SEQ_LEN)
            def _(nxt=nxt, b=b):
                fire(nxt, b)
        return 0
    lax.fori_loop(0, _NGRP, grp, 0)

    pltpu.sync_copy(acc_v, out_hbm.at[pl.ds(wid * _BPW, _BPW)])


def _detile_body(tok_ref, out_ref):
    t = tok_ref[...]
    q = _TBLK // 4
    out_ref[0] = (t // _TBLK) * _TBLK + (t % q) * 4 + (t % _TBLK) // q


_TBLK = 32768
_TCH = 128


_TGRID = pl.cdiv(VOCAB, _TBLK)
_VPAD = _TGRID * _TBLK


def _transpose_body(tab_ref, out_ref):
    q = _TBLK // 4
    x = tab_ref[...]
    eye = jnp.eye(_TCH, dtype=jnp.float32)
    parts = []
    for c in range(4):
        chunks = []
        for k in range(q // _TCH):
            xc = x[:, c * q + k * _TCH:c * q + (k + 1) * _TCH]
            yc = lax.dot_general(eye, xc, (((1,), (1,)), ((), ())))
            chunks.append(yc)
        parts.append(jnp.concatenate(chunks, axis=0))
    out_ref[...] = jnp.concatenate(parts, axis=1)


@jax.jit
def kernel(sentence_tokens, embedding_table):
    tok = sentence_tokens.astype(jnp.int32)
    tok = pl.pallas_call(
        _detile_body,
        grid=(_NW,),
        in_specs=[pl.BlockSpec((SEQ_LEN, _BPW), lambda j: (0, j))],
        out_specs=pl.BlockSpec((1, SEQ_LEN, _BPW), lambda j: (j, 0, 0)),
        out_shape=jax.ShapeDtypeStruct((_NW, SEQ_LEN, _BPW), jnp.int32),
    )(tok)
    tab_flat = pl.pallas_call(
        _transpose_body,
        grid=(_TGRID,),
        in_specs=[pl.BlockSpec((EMBED_DIM, _TBLK), lambda j: (0, j))],
        out_specs=pl.BlockSpec((_TBLK // 4, 128), lambda j: (j, 0)),
        out_shape=jax.ShapeDtypeStruct((_VPAD * EMBED_DIM // 128, 128),
                                       jnp.float32),
    )(embedding_table.T)
    table_lin = tab_flat.reshape(_VPAD, EMBED_DIM)
    mesh = plsc.VectorSubcoreMesh(core_axis_name="c", subcore_axis_name="s")
    run = functools.partial(
        pl.kernel,
        out_type=jax.ShapeDtypeStruct((BATCH, EMBED_DIM), jnp.float32),
        mesh=mesh,
        scratch_types=[
            pltpu.VMEM((SEQ_LEN, _BPW), jnp.int32),
            pltpu.VMEM((_NBUF, _BPW, EMBED_DIM), jnp.float32),
            pltpu.VMEM((_BPW, EMBED_DIM), jnp.float32),
        ] + [pltpu.SemaphoreType.DMA] * _NBUF,
        compiler_params=pltpu.CompilerParams(use_tc_tiling_on_sc=False),
    )(_embed_bag_body)
    return run(tok, table_lin)
